# baseline (speedup 1.0000x reference)
import functools

import jax
import jax.numpy as jnp
from jax import lax
from jax.experimental import pallas as pl
from jax.experimental.pallas import tpu as pltpu
from jax.experimental.pallas import tpu_sc as plsc

NF = 8
BATCH = 16384
EMBED = 64
LANES = 16

NC = 2
NS = 16
NW = NC * NS
BPW = BATCH // NW
CHUNK = 128
ROUNDS = BPW // CHUNK


def _body(i0, i1, i2, i3, i4, i5, i6, i7,
          w0, w1, w2, w3, w4, w5, w6, w7,
          out, idx_v, buf, outb, sem):
    idxs = [i0, i1, i2, i3, i4, i5, i6, i7]
    tables = [w0, w1, w2, w3, w4, w5, w6, w7]
    wid = lax.axis_index("s") * NC + lax.axis_index("c")
    base = wid * BPW

    for f in range(NF):
        pltpu.sync_copy(idxs[f].at[pl.ds(base, BPW)], idx_v.at[f])

    for r in range(ROUNDS):
        cps = [
            pltpu.async_copy(
                tables[f].at[idx_v.at[f, pl.ds(r * CHUNK, CHUNK)]],
                buf.at[f], sem)
            for f in range(NF)
        ]
        for cp in cps:
            cp.wait()

        def sum_row(i, carry):
            for c in range(EMBED // LANES):
                acc = buf[0, i, pl.ds(c * LANES, LANES)]
                for f in range(1, NF):
                    acc = acc + buf[f, i, pl.ds(c * LANES, LANES)]
                outb[i, pl.ds(c * LANES, LANES)] = acc
            return carry

        lax.fori_loop(0, CHUNK, sum_row, 0)
        pltpu.sync_copy(outb, out.at[pl.ds(base + r * CHUNK, CHUNK)])


_poly_embed = functools.partial(
    pl.kernel,
    mesh=plsc.VectorSubcoreMesh(core_axis_name="c", subcore_axis_name="s"),
    out_type=jax.ShapeDtypeStruct((BATCH, EMBED), jnp.float32),
    scratch_types=[
        pltpu.VMEM((NF, BPW), jnp.int32),
        pltpu.VMEM((NF, CHUNK, EMBED), jnp.float32),
        pltpu.VMEM((CHUNK, EMBED), jnp.float32),
        pltpu.SemaphoreType.DMA,
    ],
    compiler_params=pltpu.CompilerParams(use_tc_tiling_on_sc=False),
)(_body)


@jax.jit
def kernel(idx_0, idx_1, idx_2, idx_3, idx_4, idx_5, idx_6, idx_7,
           W_0, W_1, W_2, W_3, W_4, W_5, W_6, W_7):
    return _poly_embed(idx_0, idx_1, idx_2, idx_3, idx_4, idx_5, idx_6, idx_7,
                       W_0, W_1, W_2, W_3, W_4, W_5, W_6, W_7)
